# baseline (speedup 1.0000x reference)
import jax
import jax.numpy as jnp
from jax.experimental import pallas as pl
from jax.experimental.pallas import tpu as pltpu

_B, _H, _W, _K, _C = 1, 384, 384, 4, 96
_BH = 64
_BH2 = _BH // 2
_NSTEPS = _H // _BH


def _masked_copy(tex_hbm, p_ref, o_ref, buf, sems):
    i = pl.program_id(0)

    def in_copy(slot, step, half):
        return pltpu.make_async_copy(
            tex_hbm.at[0, pl.ds(step * _BH + half * _BH2, _BH2), 0, :, :],
            buf.at[slot, pl.ds(half * _BH2, _BH2)],
            sems.at[slot, half],
        )

    @pl.when(i == 0)
    def _():
        in_copy(0, 0, 0).start()
        in_copy(0, 0, 1).start()

    @pl.when(i + 1 < _NSTEPS)
    def _():
        in_copy((i + 1) % 2, i + 1, 0).start()
        in_copy((i + 1) % 2, i + 1, 1).start()

    in_copy(i % 2, i, 0).wait()
    in_copy(i % 2, i, 1).wait()
    mask = p_ref[0, :, 0:1, :] >= 0
    o_ref[0, :, :, :] = jnp.where(mask, buf[i % 2], 0.0)


def kernel(texels, pix_to_face):
    tex_t = texels.transpose(0, 1, 3, 4, 2)
    pix_t = pix_to_face.transpose(0, 1, 3, 2)
    out_t = pl.pallas_call(
        _masked_copy,
        grid=(_NSTEPS,),
        in_specs=[
            pl.BlockSpec(memory_space=pltpu.MemorySpace.HBM),
            pl.BlockSpec((1, _BH, _K, _W), lambda i: (0, i, 0, 0)),
        ],
        out_specs=pl.BlockSpec((1, _BH, _C, _W), lambda i: (0, i, 0, 0)),
        out_shape=jax.ShapeDtypeStruct((_B, _H, _C, _W), jnp.float32),
        scratch_shapes=[
            pltpu.VMEM((2, _BH, _C, _W), jnp.float32),
            pltpu.SemaphoreType.DMA((2, 2)),
        ],
        compiler_params=pltpu.CompilerParams(
            dimension_semantics=("arbitrary",),
        ),
    )(tex_t, pix_t)
    return out_t.transpose(0, 1, 3, 2)


# speedup vs baseline: 1.0359x; 1.0359x over previous
import jax
import jax.numpy as jnp
from jax.experimental import pallas as pl
from jax.experimental.pallas import tpu as pltpu

_B, _H, _W, _K, _C = 1, 384, 384, 4, 96
_BH = 96


def _masked_copy(tex_ref, p_ref, o_ref):
    mask = p_ref[0, :, 0:1, :] >= 0
    o_ref[0, :, :, :] = jnp.where(mask, tex_ref[0, :, 0, :, :], 0.0)


def kernel(texels, pix_to_face):
    tex_t = texels.transpose(0, 1, 3, 4, 2)
    pix_t = pix_to_face.transpose(0, 1, 3, 2)
    out_t = pl.pallas_call(
        _masked_copy,
        grid=(_H // _BH,),
        in_specs=[
            pl.BlockSpec((1, _BH, 1, _C, _W), lambda i: (0, i, 0, 0, 0)),
            pl.BlockSpec((1, _BH, _K, _W), lambda i: (0, i, 0, 0)),
        ],
        out_specs=pl.BlockSpec((1, _BH, _C, _W), lambda i: (0, i, 0, 0)),
        out_shape=jax.ShapeDtypeStruct((_B, _H, _C, _W), jnp.float32),
        compiler_params=pltpu.CompilerParams(
            dimension_semantics=("arbitrary",),
        ),
    )(tex_t, pix_t)
    return out_t.transpose(0, 1, 3, 2)
